# Initial kernel scaffold; baseline (speedup 1.0000x reference)
import jax
import jax.numpy as jnp
from jax.experimental import pallas as pl


def kernel(xyz1, xyz2, feat1, feat2, value1, W1, b1, W2, b2):
    raise NotImplementedError("write your pallas kernel here")


# speedup vs baseline: 12.6717x; 12.6717x over previous
import functools

import jax
import jax.numpy as jnp
from jax.experimental import pallas as pl
from jax.experimental.pallas import tpu as pltpu

NSAMPLE = 16
C = 256
B, N1, N2 = 4, 1024, 4096
Q = 256
NB = N2 // Q


def _dot(a, b):
    return jax.lax.dot_general(a, b, (((1,), (0,)), ((), ())),
                               preferred_element_type=jnp.float32)


def _prep_body(feat1_ref, feat2_ref, w1a_ref, w1b_ref, b1_ref,
               a1t_ref, a2t_ref):
    f1 = feat1_ref[0]
    f2 = feat2_ref[0]
    w1a = w1a_ref[...]
    w1b = w1b_ref[...]
    a1t_ref[0] = jax.lax.dot_general(
        f1, w1a, (((0,), (1,)), ((), ())), preferred_element_type=jnp.float32)
    a2t_ref[0] = jax.lax.dot_general(
        f2, w1b, (((0,), (1,)), ((), ())),
        preferred_element_type=jnp.float32) + b1_ref[...]


def _main_body(xyz1_ref, xyz2t_ref, xyz1t_ref, a1t_ref, a2t_ref, vt_ref,
               w1ct_ref, w2t_ref, b2_ref, out_ref):
    q = xyz2t_ref[0]
    t = xyz1_ref[0]
    d = jnp.zeros((Q, N1), jnp.float32)
    for dim in range(3):
        diff = q[:, dim:dim + 1] - t[dim:dim + 1, :]
        d = d + diff * diff

    iota = jax.lax.broadcasted_iota(jnp.int32, (Q, N1), 1)
    a1t = a1t_ref[0]
    vt = vt_ref[0]
    x1t = xyz1t_ref[0]
    a2 = a2t_ref[0]
    w1ct = w1ct_ref[...]
    w2t = w2t_ref[...]
    b2 = b2_ref[...]

    ws = []
    gvs = []
    for _k in range(NSAMPLE):
        m = jnp.min(d, axis=1, keepdims=True)
        idx = jnp.min(jnp.where(d == m, iota, N1 + 1), axis=1,
                      keepdims=True)
        onehot = (iota == idx).astype(jnp.float32)
        d = jnp.where(iota == idx, jnp.inf, d)
        ga = _dot(onehot, a1t)
        gv = _dot(onehot, vt)
        gx = _dot(onehot, x1t)
        h = ga + a2 + _dot(gx - q, w1ct)
        h = jnp.where(h >= 0, h, 0.1 * h)
        ws.append(_dot(h, w2t) + b2)
        gvs.append(gv)

    w = jnp.stack(ws)
    gv = jnp.stack(gvs)
    wmax = jnp.max(w, axis=0, keepdims=True)
    e = jnp.exp(w - wmax)
    out_ref[0] = jnp.sum(e * gv, axis=0) / jnp.sum(e, axis=0)


@jax.jit
def _run(xyz1, xyz2, feat1, feat2, value1, W1, b1, W2, b2):
    w1a = W1[:, :C]
    w1b = W1[:, C:2 * C]
    w1ct = jnp.zeros((8, C), jnp.float32).at[:3, :].set(W1[:, 2 * C:].T)
    xyz1t = jnp.concatenate(
        [jnp.transpose(xyz1, (0, 2, 1)),
         jnp.zeros((B, N1, 5), jnp.float32)], axis=-1)
    xyz2t = jnp.concatenate(
        [jnp.transpose(xyz2, (0, 2, 1)),
         jnp.zeros((B, N2, 5), jnp.float32)], axis=-1)
    vt = jnp.transpose(value1, (0, 2, 1))

    a1t, a2t = pl.pallas_call(
        _prep_body,
        grid=(B,),
        in_specs=[
            pl.BlockSpec((1, C, N1), lambda b: (b, 0, 0)),
            pl.BlockSpec((1, C, N2), lambda b: (b, 0, 0)),
            pl.BlockSpec((C, C), lambda b: (0, 0)),
            pl.BlockSpec((C, C), lambda b: (0, 0)),
            pl.BlockSpec((1, C), lambda b: (0, 0)),
        ],
        out_specs=[
            pl.BlockSpec((1, N1, C), lambda b: (b, 0, 0)),
            pl.BlockSpec((1, N2, C), lambda b: (b, 0, 0)),
        ],
        out_shape=[
            jax.ShapeDtypeStruct((B, N1, C), jnp.float32),
            jax.ShapeDtypeStruct((B, N2, C), jnp.float32),
        ],
    )(feat1, feat2, w1a, w1b, b1[None, :])

    outt = pl.pallas_call(
        _main_body,
        grid=(B, NB),
        in_specs=[
            pl.BlockSpec((1, 3, N1), lambda b, i: (b, 0, 0)),
            pl.BlockSpec((1, Q, 8), lambda b, i: (b, i, 0)),
            pl.BlockSpec((1, N1, 8), lambda b, i: (b, 0, 0)),
            pl.BlockSpec((1, N1, C), lambda b, i: (b, 0, 0)),
            pl.BlockSpec((1, Q, C), lambda b, i: (b, i, 0)),
            pl.BlockSpec((1, N1, C), lambda b, i: (b, 0, 0)),
            pl.BlockSpec((8, C), lambda b, i: (0, 0)),
            pl.BlockSpec((C, C), lambda b, i: (0, 0)),
            pl.BlockSpec((1, C), lambda b, i: (0, 0)),
        ],
        out_specs=pl.BlockSpec((1, Q, C), lambda b, i: (b, i, 0)),
        out_shape=jax.ShapeDtypeStruct((B, N2, C), jnp.float32),
    )(xyz1, xyz2t, xyz1t, a1t, a2t, vt, w1ct, W2.T, b2[None, :])

    return jnp.transpose(outt, (0, 2, 1))


def kernel(xyz1, xyz2, feat1, feat2, value1, W1, b1, W2, b2):
    return _run(xyz1, xyz2, feat1, feat2, value1, W1, b1, W2, b2)


# speedup vs baseline: 19.0764x; 1.5054x over previous
import functools

import jax
import jax.numpy as jnp
from jax.experimental import pallas as pl
from jax.experimental.pallas import tpu as pltpu

NSAMPLE = 16
C = 256
B, N1, N2 = 4, 1024, 4096
Q = 256
NB = N2 // Q


def _dot(a, b):
    return jax.lax.dot_general(a, b, (((1,), (0,)), ((), ())),
                               preferred_element_type=jnp.float32)


def _prep_body(feat1_ref, feat2_ref, x1t_ref, x2t_ref, w1a_ref, w1b_ref,
               w1ct_ref, b1_ref, a1t_ref, a2t_ref):
    f1 = feat1_ref[0]
    f2 = feat2_ref[0]
    w1ct = w1ct_ref[...]
    a1t_ref[0] = jax.lax.dot_general(
        f1, w1a_ref[...], (((0,), (1,)), ((), ())),
        preferred_element_type=jnp.float32) + _dot(x1t_ref[0], w1ct)
    a2t_ref[0] = (jax.lax.dot_general(
        f2, w1b_ref[...], (((0,), (1,)), ((), ())),
        preferred_element_type=jnp.float32)
        + b1_ref[...] - _dot(x2t_ref[0], w1ct))


def _main_body(xyz1_ref, xyz2t_ref, a1t_ref, a2t_ref, vt_ref,
               w2t_ref, b2_ref, out_ref):
    q = xyz2t_ref[0]
    t = xyz1_ref[0]
    d = jnp.zeros((Q, N1), jnp.float32)
    for dim in range(3):
        diff = q[:, dim:dim + 1] - t[dim:dim + 1, :]
        d = d + diff * diff

    iota = jax.lax.broadcasted_iota(jnp.int32, (Q, N1), 1)
    a1t = a1t_ref[0]
    vt = vt_ref[0]
    a2 = a2t_ref[0]
    w2t = w2t_ref[...]
    b2 = b2_ref[...]

    num = jnp.zeros((Q, C), jnp.float32)
    den = jnp.zeros((Q, C), jnp.float32)
    for _k in range(NSAMPLE):
        m = jnp.min(d, axis=1, keepdims=True)
        idx = jnp.min(jnp.where(d == m, iota, N1 + 1), axis=1,
                      keepdims=True)
        onehot = (iota == idx).astype(jnp.float32)
        d = jnp.where(iota == idx, jnp.inf, d)
        h = _dot(onehot, a1t) + a2
        h = jnp.maximum(h, 0.1 * h)
        e = jnp.exp(_dot(h, w2t) + b2)
        num = num + e * _dot(onehot, vt)
        den = den + e
    out_ref[0] = num / den


@jax.jit
def _run(xyz1, xyz2, feat1, feat2, value1, W1, b1, W2, b2):
    w1a = W1[:, :C]
    w1b = W1[:, C:2 * C]
    w1ct = jnp.zeros((8, C), jnp.float32).at[:3, :].set(W1[:, 2 * C:].T)
    xyz1t = jnp.concatenate(
        [jnp.transpose(xyz1, (0, 2, 1)),
         jnp.zeros((B, N1, 5), jnp.float32)], axis=-1)
    xyz2t = jnp.concatenate(
        [jnp.transpose(xyz2, (0, 2, 1)),
         jnp.zeros((B, N2, 5), jnp.float32)], axis=-1)
    vt = jnp.transpose(value1, (0, 2, 1))

    a1t, a2t = pl.pallas_call(
        _prep_body,
        grid=(B,),
        in_specs=[
            pl.BlockSpec((1, C, N1), lambda b: (b, 0, 0)),
            pl.BlockSpec((1, C, N2), lambda b: (b, 0, 0)),
            pl.BlockSpec((1, N1, 8), lambda b: (b, 0, 0)),
            pl.BlockSpec((1, N2, 8), lambda b: (b, 0, 0)),
            pl.BlockSpec((C, C), lambda b: (0, 0)),
            pl.BlockSpec((C, C), lambda b: (0, 0)),
            pl.BlockSpec((8, C), lambda b: (0, 0)),
            pl.BlockSpec((1, C), lambda b: (0, 0)),
        ],
        out_specs=[
            pl.BlockSpec((1, N1, C), lambda b: (b, 0, 0)),
            pl.BlockSpec((1, N2, C), lambda b: (b, 0, 0)),
        ],
        out_shape=[
            jax.ShapeDtypeStruct((B, N1, C), jnp.float32),
            jax.ShapeDtypeStruct((B, N2, C), jnp.float32),
        ],
    )(feat1, feat2, xyz1t, xyz2t, w1a, w1b, w1ct, b1[None, :])

    outt = pl.pallas_call(
        _main_body,
        grid=(B, NB),
        in_specs=[
            pl.BlockSpec((1, 3, N1), lambda b, i: (b, 0, 0)),
            pl.BlockSpec((1, Q, 8), lambda b, i: (b, i, 0)),
            pl.BlockSpec((1, N1, C), lambda b, i: (b, 0, 0)),
            pl.BlockSpec((1, Q, C), lambda b, i: (b, i, 0)),
            pl.BlockSpec((1, N1, C), lambda b, i: (b, 0, 0)),
            pl.BlockSpec((C, C), lambda b, i: (0, 0)),
            pl.BlockSpec((1, C), lambda b, i: (0, 0)),
        ],
        out_specs=pl.BlockSpec((1, Q, C), lambda b, i: (b, i, 0)),
        out_shape=jax.ShapeDtypeStruct((B, N2, C), jnp.float32),
    )(xyz1, xyz2t, a1t, a2t, vt, W2.T, b2[None, :])

    return jnp.transpose(outt, (0, 2, 1))


def kernel(xyz1, xyz2, feat1, feat2, value1, W1, b1, W2, b2):
    return _run(xyz1, xyz2, feat1, feat2, value1, W1, b1, W2, b2)


# speedup vs baseline: 23.0512x; 1.2084x over previous
import functools

import jax
import jax.numpy as jnp
from jax.experimental import pallas as pl
from jax.experimental.pallas import tpu as pltpu

NSAMPLE = 16
C = 256
B, N1, N2 = 4, 1024, 4096
Q = 256
NB = N2 // Q


def _dot(a, b):
    return jax.lax.dot_general(a, b, (((1,), (0,)), ((), ())),
                               preferred_element_type=jnp.float32)


def _prep_body(feat1_ref, feat2_ref, value1_ref, x1t_ref, x2t_ref, w1a_ref,
               w1b_ref, w1ct_ref, b1_ref, tab_ref, a2t_ref):
    f1 = feat1_ref[0]
    f2 = feat2_ref[0]
    w1ct = w1ct_ref[...]
    a1 = jax.lax.dot_general(
        f1, w1a_ref[...], (((0,), (1,)), ((), ())),
        preferred_element_type=jnp.float32) + _dot(x1t_ref[0], w1ct)
    vt = jnp.transpose(value1_ref[0], (1, 0))
    tab_ref[0] = jnp.concatenate([a1, vt], axis=1).astype(jnp.bfloat16)
    a2t_ref[0] = (jax.lax.dot_general(
        f2, w1b_ref[...], (((0,), (1,)), ((), ())),
        preferred_element_type=jnp.float32)
        + b1_ref[...] - _dot(x2t_ref[0], w1ct))


def _main_body(xyz1_ref, xyz2t_ref, tab_ref, a2t_ref, w2t_ref, b2_ref,
               out_ref):
    q = xyz2t_ref[0]
    t = xyz1_ref[0]
    d = jnp.zeros((Q, N1), jnp.float32)
    for dim in range(3):
        diff = q[:, dim:dim + 1] - t[dim:dim + 1, :]
        d = d + diff * diff

    tab = tab_ref[0]
    a2 = a2t_ref[0]
    w2t = w2t_ref[...]
    b2 = b2_ref[...]

    num = jnp.zeros((Q, C), jnp.float32)
    den = jnp.zeros((Q, C), jnp.float32)
    for _k in range(NSAMPLE):
        m = jnp.min(d, axis=1, keepdims=True)
        eq = d == m
        onehot = eq.astype(jnp.bfloat16)
        d = jnp.where(eq, jnp.inf, d)
        g = _dot(onehot, tab)
        h = g[:, :C] + a2
        h = jnp.maximum(h, 0.1 * h)
        e = jnp.exp(_dot(h.astype(jnp.bfloat16), w2t) + b2)
        num = num + e * g[:, C:]
        den = den + e
    out_ref[0] = num / den


@jax.jit
def _run(xyz1, xyz2, feat1, feat2, value1, W1, b1, W2, b2):
    w1a = W1[:, :C]
    w1b = W1[:, C:2 * C]
    w1ct = jnp.zeros((8, C), jnp.float32).at[:3, :].set(W1[:, 2 * C:].T)
    xyz1t = jnp.concatenate(
        [jnp.transpose(xyz1, (0, 2, 1)),
         jnp.zeros((B, N1, 5), jnp.float32)], axis=-1)
    xyz2t = jnp.concatenate(
        [jnp.transpose(xyz2, (0, 2, 1)),
         jnp.zeros((B, N2, 5), jnp.float32)], axis=-1)

    tab, a2t = pl.pallas_call(
        _prep_body,
        grid=(B,),
        in_specs=[
            pl.BlockSpec((1, C, N1), lambda b: (b, 0, 0)),
            pl.BlockSpec((1, C, N2), lambda b: (b, 0, 0)),
            pl.BlockSpec((1, C, N1), lambda b: (b, 0, 0)),
            pl.BlockSpec((1, N1, 8), lambda b: (b, 0, 0)),
            pl.BlockSpec((1, N2, 8), lambda b: (b, 0, 0)),
            pl.BlockSpec((C, C), lambda b: (0, 0)),
            pl.BlockSpec((C, C), lambda b: (0, 0)),
            pl.BlockSpec((8, C), lambda b: (0, 0)),
            pl.BlockSpec((1, C), lambda b: (0, 0)),
        ],
        out_specs=[
            pl.BlockSpec((1, N1, 2 * C), lambda b: (b, 0, 0)),
            pl.BlockSpec((1, N2, C), lambda b: (b, 0, 0)),
        ],
        out_shape=[
            jax.ShapeDtypeStruct((B, N1, 2 * C), jnp.bfloat16),
            jax.ShapeDtypeStruct((B, N2, C), jnp.float32),
        ],
    )(feat1, feat2, value1, xyz1t, xyz2t, w1a, w1b, w1ct, b1[None, :])

    outt = pl.pallas_call(
        _main_body,
        grid=(B, NB),
        in_specs=[
            pl.BlockSpec((1, 3, N1), lambda b, i: (b, 0, 0)),
            pl.BlockSpec((1, Q, 8), lambda b, i: (b, i, 0)),
            pl.BlockSpec((1, N1, 2 * C), lambda b, i: (b, 0, 0)),
            pl.BlockSpec((1, Q, C), lambda b, i: (b, i, 0)),
            pl.BlockSpec((C, C), lambda b, i: (0, 0)),
            pl.BlockSpec((1, C), lambda b, i: (0, 0)),
        ],
        out_specs=pl.BlockSpec((1, Q, C), lambda b, i: (b, i, 0)),
        out_shape=jax.ShapeDtypeStruct((B, N2, C), jnp.float32),
    )(xyz1, xyz2t, tab, a2t, W2.T.astype(jnp.bfloat16), b2[None, :])

    return jnp.transpose(outt, (0, 2, 1))


def kernel(xyz1, xyz2, feat1, feat2, value1, W1, b1, W2, b2):
    return _run(xyz1, xyz2, feat1, feat2, value1, W1, b1, W2, b2)


# speedup vs baseline: 25.8595x; 1.1218x over previous
import functools

import jax
import jax.numpy as jnp
from jax.experimental import pallas as pl
from jax.experimental.pallas import tpu as pltpu

NSAMPLE = 16
C = 256
B, N1, N2 = 4, 1024, 4096
Q = 512
NB = N2 // Q


def _dot(a, b):
    return jax.lax.dot_general(a, b, (((1,), (0,)), ((), ())),
                               preferred_element_type=jnp.float32)


def _prep_body(feat1_ref, feat2_ref, value1_ref, x1t_ref, x2t_ref, w1a_ref,
               w1b_ref, w1ct_ref, b1_ref, tab_ref, a2t_ref):
    f1 = feat1_ref[0]
    f2 = feat2_ref[0]
    w1ct = w1ct_ref[...]
    a1 = jax.lax.dot_general(
        f1, w1a_ref[...], (((0,), (1,)), ((), ())),
        preferred_element_type=jnp.float32) + _dot(x1t_ref[0], w1ct)
    vt = jnp.transpose(value1_ref[0], (1, 0))
    tab_ref[0] = jnp.concatenate([a1, vt], axis=1).astype(jnp.bfloat16)
    a2t_ref[0] = (jax.lax.dot_general(
        f2, w1b_ref[...], (((0,), (1,)), ((), ())),
        preferred_element_type=jnp.float32)
        + b1_ref[...] - _dot(x2t_ref[0], w1ct))


def _main_body(xyz1_ref, xyz2t_ref, tab_ref, a2t_ref, w2t_ref, b2_ref,
               out_ref):
    q = xyz2t_ref[0]
    t = xyz1_ref[0]
    d = jnp.zeros((Q, N1), jnp.float32)
    for dim in range(3):
        diff = q[:, dim:dim + 1] - t[dim:dim + 1, :]
        d = d + diff * diff

    tab = tab_ref[0]
    a2 = a2t_ref[0]
    w2t = w2t_ref[...]
    b2 = b2_ref[...]

    num = jnp.zeros((Q, C), jnp.float32)
    den = jnp.zeros((Q, C), jnp.float32)
    for _k in range(NSAMPLE):
        m = jnp.min(d, axis=1, keepdims=True)
        eq = d == m
        onehot = eq.astype(jnp.bfloat16)
        d = jnp.where(eq, jnp.inf, d)
        g = _dot(onehot, tab)
        h = g[:, :C] + a2
        h = jnp.maximum(h, 0.1 * h)
        e = jnp.exp(_dot(h.astype(jnp.bfloat16), w2t) + b2)
        num = num + e * g[:, C:]
        den = den + e
    out_ref[0] = jnp.transpose(num / den, (1, 0))


@jax.jit
def _run(xyz1, xyz2, feat1, feat2, value1, W1, b1, W2, b2):
    w1a = W1[:, :C]
    w1b = W1[:, C:2 * C]
    w1ct = jnp.zeros((8, C), jnp.float32).at[:3, :].set(W1[:, 2 * C:].T)
    xyz1t = jnp.concatenate(
        [jnp.transpose(xyz1, (0, 2, 1)),
         jnp.zeros((B, N1, 5), jnp.float32)], axis=-1)
    xyz2t = jnp.concatenate(
        [jnp.transpose(xyz2, (0, 2, 1)),
         jnp.zeros((B, N2, 5), jnp.float32)], axis=-1)

    tab, a2t = pl.pallas_call(
        _prep_body,
        grid=(B,),
        in_specs=[
            pl.BlockSpec((1, C, N1), lambda b: (b, 0, 0)),
            pl.BlockSpec((1, C, N2), lambda b: (b, 0, 0)),
            pl.BlockSpec((1, C, N1), lambda b: (b, 0, 0)),
            pl.BlockSpec((1, N1, 8), lambda b: (b, 0, 0)),
            pl.BlockSpec((1, N2, 8), lambda b: (b, 0, 0)),
            pl.BlockSpec((C, C), lambda b: (0, 0)),
            pl.BlockSpec((C, C), lambda b: (0, 0)),
            pl.BlockSpec((8, C), lambda b: (0, 0)),
            pl.BlockSpec((1, C), lambda b: (0, 0)),
        ],
        out_specs=[
            pl.BlockSpec((1, N1, 2 * C), lambda b: (b, 0, 0)),
            pl.BlockSpec((1, N2, C), lambda b: (b, 0, 0)),
        ],
        out_shape=[
            jax.ShapeDtypeStruct((B, N1, 2 * C), jnp.bfloat16),
            jax.ShapeDtypeStruct((B, N2, C), jnp.float32),
        ],
    )(feat1, feat2, value1, xyz1t, xyz2t, w1a, w1b, w1ct, b1[None, :])

    outt = pl.pallas_call(
        _main_body,
        grid=(B, NB),
        in_specs=[
            pl.BlockSpec((1, 3, N1), lambda b, i: (b, 0, 0)),
            pl.BlockSpec((1, Q, 8), lambda b, i: (b, i, 0)),
            pl.BlockSpec((1, N1, 2 * C), lambda b, i: (b, 0, 0)),
            pl.BlockSpec((1, Q, C), lambda b, i: (b, i, 0)),
            pl.BlockSpec((C, C), lambda b, i: (0, 0)),
            pl.BlockSpec((1, C), lambda b, i: (0, 0)),
        ],
        out_specs=pl.BlockSpec((1, C, Q), lambda b, i: (b, 0, i)),
        out_shape=jax.ShapeDtypeStruct((B, C, N2), jnp.float32),
    )(xyz1, xyz2t, tab, a2t, W2.T.astype(jnp.bfloat16), b2[None, :])

    return outt


def kernel(xyz1, xyz2, feat1, feat2, value1, W1, b1, W2, b2):
    return _run(xyz1, xyz2, feat1, feat2, value1, W1, b1, W2, b2)
